# baseline (speedup 1.0000x reference)
import functools

import jax
import jax.numpy as jnp
from jax import lax
from jax.experimental import pallas as pl
from jax.experimental.pallas import tpu as pltpu
from jax.experimental.pallas import tpu_sc as plsc

N = 8192
M = 126
SIZE = M + 2
NC, NS, L = 2, 16, 16
NW = NC * NS
RPW = N // NW
G = 8
NG = RPW // G
NCH = SIZE // L

_NEG_INF = float("-inf")
_BIG = 1 << 30


def _sc_body(srco, dsto, srcp, dstp, uavo, uavp, out_hbm,
             uav_v, uavp_v,
             srco_v, srcp_v, dsto_v, dstp_v,
             nmm_v, trace_v, md2_v, dstsc_v):
    wid = lax.axis_index("s") * NC + lax.axis_index("c")
    base_row = wid * RPW

    pltpu.sync_copy(uavo, uav_v)
    pltpu.sync_copy(uavp, uavp_v)
    for coord in range(3):
        pltpu.sync_copy(srco.at[pl.ds(coord * N + base_row, RPW)],
                        srco_v.at[pl.ds(coord * RPW, RPW)])
        pltpu.sync_copy(dsto.at[pl.ds(coord * N + base_row, RPW)],
                        dsto_v.at[pl.ds(coord * RPW, RPW)])
        pltpu.sync_copy(srcp.at[pl.ds(coord * N + base_row, RPW)],
                        srcp_v.at[pl.ds(coord * RPW, RPW)])
        pltpu.sync_copy(dstp.at[pl.ds(coord * N + base_row, RPW)],
                        dstp_v.at[pl.ds(coord * RPW, RPW)])

    iota = lax.iota(jnp.int32, L)
    neginf_v = jnp.full((L,), _NEG_INF, jnp.float32)
    zero_v = jnp.zeros((L,), jnp.float32)
    lane15 = iota == (L - 1)
    lane0 = iota == 0
    lane_lo = iota < G

    def group_body(g, _):
        rows = [g * G + r for r in range(G)]
        rowv = [jnp.full((L,), rows[r], jnp.int32) for r in range(G)]
        dgx = [plsc.load_gather(dstp_v, [rowv[r]]) for r in range(G)]
        dgy = [plsc.load_gather(dstp_v, [rowv[r] + RPW]) for r in range(G)]
        dgz = [plsc.load_gather(dstp_v, [rowv[r] + 2 * RPW]) for r in range(G)]
        for r in range(G):
            dstsc_v[0, r] = dgx[r]
            dstsc_v[1, r] = dgy[r]
            dstsc_v[2, r] = dgz[r]
        px0 = [plsc.load_gather(srcp_v, [rowv[r]]) for r in range(G)]
        py0 = [plsc.load_gather(srcp_v, [rowv[r] + RPW]) for r in range(G)]
        pz0 = [plsc.load_gather(srcp_v, [rowv[r] + 2 * RPW]) for r in range(G)]
        mask0 = jnp.where(lane0, _NEG_INF, 0.0).astype(jnp.float32)
        for c in range(NCH):
            mc = mask0 if c == 0 else zero_v
            for r in range(G):
                nmm_v[r, pl.ds(c * L, L)] = mc

        def make_step(first):
            def step(k, st):
                xp, yp, zp = st
                kv = jnp.full((L,), k * L, jnp.int32)
                bestv = [None] * G
                besti = [None] * G
                for c in range(NCH):
                    cx = uavp_v[pl.ds(c * L, L)]
                    cy = uavp_v[pl.ds(SIZE + c * L, L)]
                    cz = uavp_v[pl.ds(2 * SIZE + c * L, L)]
                    for r in range(G):
                        ddx = cx - xp[r]
                        ddy = cy - yp[r]
                        ddz = cz - zp[r]
                        d2 = ddx * ddx + ddy * ddy + ddz * ddz
                        s = nmm_v[r, pl.ds(c * L, L)] - d2
                        if c == NCH - 1:
                            if first:
                                s = jnp.where(lane15, neginf_v, s)
                            else:
                                fx = dstsc_v[0, r] - xp[r]
                                fy = dstsc_v[1, r] - yp[r]
                                fz = dstsc_v[2, r] - zp[r]
                                sd = -(fx * fx + fy * fy + fz * fz)
                                s = jnp.where(lane15, sd, s)
                        if c == 0:
                            bestv[r] = s
                            besti[r] = jnp.int32(0)
                        else:
                            gt = s > bestv[r]
                            bestv[r] = jnp.maximum(s, bestv[r])
                            besti[r] = jnp.where(gt, jnp.int32(c), besti[r])
                nxp, nyp, nzp = list(xp), list(yp), list(zp)
                for r in range(G):
                    mval = jnp.max(bestv[r])
                    gidx = besti[r] * L + iota
                    idx = jnp.min(jnp.where(bestv[r] == mval, gidx, _BIG))
                    idxv = jnp.full((L,), idx, jnp.int32)
                    nxp[r] = plsc.load_gather(uavp_v, [idxv])
                    nyp[r] = plsc.load_gather(uavp_v, [idxv + SIZE])
                    nzp[r] = plsc.load_gather(uavp_v, [idxv + 2 * SIZE])
                    plsc.store_scatter(trace_v, [kv + r], idxv, mask=lane0)
                    plsc.store_scatter(nmm_v.at[r], [idxv], neginf_v,
                                       mask=lane0 & (idxv != SIZE - 1))
                return (tuple(nxp), tuple(nyp), tuple(nzp))
            return step

        st0 = make_step(True)(0, (tuple(px0), tuple(py0), tuple(pz0)))
        lax.fori_loop(1, SIZE, make_step(False), st0)

        rlane = jnp.int32(g * G) + jnp.where(lane_lo, iota, 0)
        dox = plsc.load_gather(dsto_v, [rlane])
        doy = plsc.load_gather(dsto_v, [rlane + RPW])
        doz = plsc.load_gather(dsto_v, [rlane + 2 * RPW])
        ox0 = plsc.load_gather(srco_v, [rlane])
        oy0 = plsc.load_gather(srco_v, [rlane + RPW])
        oz0 = plsc.load_gather(srco_v, [rlane + 2 * RPW])

        def replay(k, st):
            xo, yo, zo, md2, done = st
            idxk = trace_v[pl.ds(k * L, L)]
            is_dst = idxk == (SIZE - 1)
            hx = plsc.load_gather(uav_v, [idxk], mask=lane_lo)
            hy = plsc.load_gather(uav_v, [idxk + SIZE], mask=lane_lo)
            hz = plsc.load_gather(uav_v, [idxk + 2 * SIZE], mask=lane_lo)
            nxo = jnp.where(is_dst, dox, hx)
            nyo = jnp.where(is_dst, doy, hy)
            nzo = jnp.where(is_dst, doz, hz)
            ex = nxo - xo
            ey = nyo - yo
            ez = nzo - zo
            dd2 = ex * ex + ey * ey + ez * ez
            nmd = jnp.where(done, md2, jnp.maximum(md2, dd2))
            return (nxo, nyo, nzo, nmd, done | is_dst)

        st2 = lax.fori_loop(0, SIZE, replay,
                            (ox0, oy0, oz0, zero_v,
                             jnp.zeros((L,), jnp.bool_)))
        plsc.store_scatter(md2_v, [rlane], st2[3], mask=lane_lo)
        return 0

    lax.fori_loop(0, NG, group_body, 0)
    pltpu.sync_copy(md2_v, out_hbm.at[pl.ds(base_row, RPW)])


_sc_path = functools.partial(
    pl.kernel,
    out_type=jax.ShapeDtypeStruct((N,), jnp.float32),
    mesh=plsc.VectorSubcoreMesh(core_axis_name="c", subcore_axis_name="s"),
    compiler_params=pltpu.CompilerParams(needs_layout_passes=False),
    scratch_types=[
        pltpu.VMEM((3 * SIZE,), jnp.float32),
        pltpu.VMEM((3 * SIZE,), jnp.float32),
        pltpu.VMEM((3 * RPW,), jnp.float32),
        pltpu.VMEM((3 * RPW,), jnp.float32),
        pltpu.VMEM((3 * RPW,), jnp.float32),
        pltpu.VMEM((3 * RPW,), jnp.float32),
        pltpu.VMEM((G, SIZE), jnp.float32),
        pltpu.VMEM((SIZE * L,), jnp.int32),
        pltpu.VMEM((RPW,), jnp.float32),
        pltpu.VMEM((3, G, L), jnp.float32),
    ],
)(_sc_body)


def _mean_sqrt_body(x_ref, o_ref):
    o_ref[0, 0] = jnp.sum(jnp.sqrt(x_ref[...])) * jnp.float32(1.0 / N)


_mean_sqrt = pl.pallas_call(
    _mean_sqrt_body,
    out_shape=jax.ShapeDtypeStruct((1, 1), jnp.float32),
    out_specs=pl.BlockSpec(memory_space=pltpu.SMEM),
)


def kernel(outputs, W):
    src = outputs[:N]
    dst = outputs[N:2 * N]
    uav = outputs[2 * N:]
    srcp = src @ W
    dstp = dst @ W
    uavp = uav @ W
    srco = src.T.reshape(-1)
    dsto = dst.T.reshape(-1)
    srcp_t = srcp.T.reshape(-1)
    dstp_t = dstp.T.reshape(-1)
    uavo = jnp.zeros((3, SIZE), jnp.float32).at[:, 1:SIZE - 1].set(uav.T).reshape(-1)
    uavp_t = jnp.zeros((3, SIZE), jnp.float32).at[:, 1:SIZE - 1].set(uavp.T).reshape(-1)
    md2 = _sc_path(srco, dsto, srcp_t, dstp_t, uavo, uavp_t)
    return _mean_sqrt(md2.reshape(N // 128, 128))[0, 0]
